# Initial kernel scaffold; baseline (speedup 1.0000x reference)
import jax
import jax.numpy as jnp
from jax.experimental import pallas as pl


def kernel(content, salience, workspace, workspace_mask, W, b):
    raise NotImplementedError("write your pallas kernel here")


# speedup vs baseline: 2.1122x; 2.1122x over previous
import jax
import jax.numpy as jnp
from jax.experimental import pallas as pl
from jax.experimental.pallas import tpu as pltpu

_CAP = 65536
_D = 512
_BLK = 4096
_GRID = _CAP // _BLK
_BIG_I = 2 ** 30


def _tc_body(mask2d_ref, sal_ref, wrow_ref, ws_ref, content_ref, W_ref, b_ref,
             out_ref, acc_ref, idx_ref, den_ref):
    i = pl.program_id(0)

    @pl.when(i == 0)
    def _stats():
        m = mask2d_ref[...]
        r = jax.lax.broadcasted_iota(jnp.int32, m.shape, 0)
        c = jax.lax.broadcasted_iota(jnp.int32, m.shape, 1)
        j = r * 128 + c
        avail = m < 0.5
        cand = jnp.where(avail, j, _BIG_I)
        first_avail = jnp.min(cand)
        has_avail = first_avail < _BIG_I
        mmin = jnp.min(m)
        idx_min = jnp.min(jnp.where(m == mmin, j, _BIG_I))
        idx = jnp.where(has_avail, first_avail, idx_min)
        em = jnp.exp(m)
        sum_exp = jnp.sum(em)
        exp_at_idx = jnp.sum(jnp.where(j == idx, em, 0.0))
        idx_ref[0] = idx
        den_ref[0] = sum_exp - exp_at_idx
        acc_ref[...] = jnp.zeros_like(acc_ref)

    idx = idx_ref[0]
    w = jnp.exp(wrow_ref[...])
    gl = jax.lax.broadcasted_iota(jnp.int32, w.shape, 1) + i * _BLK
    w = jnp.where(gl == idx, 0.0, w)
    acc_ref[...] += jax.lax.dot_general(
        w, ws_ref[...], (((1,), (0,)), ((), ())),
        preferred_element_type=jnp.float32,
        precision=jax.lax.Precision.HIGHEST)

    @pl.when(i == _GRID - 1)
    def _final():
        es = jnp.exp(sal_ref[...])
        denom = den_ref[0] + es
        bcast = (acc_ref[...] + es * content_ref[...]) / denom
        out_ref[...] = jax.lax.dot_general(
            bcast, W_ref[...], (((1,), (1,)), ((), ())),
            preferred_element_type=jnp.float32,
            precision=jax.lax.Precision.HIGHEST) + b_ref[...]


def _run(content, salience, workspace, workspace_mask, W, b, interpret=False):
    mask2d = workspace_mask.reshape(512, 128)
    wrow = workspace_mask.reshape(1, _CAP)
    sal = salience.reshape(1, 1)
    cont = content.reshape(1, _D)
    bb = b.reshape(1, _D)
    out = pl.pallas_call(
        _tc_body,
        grid=(_GRID,),
        in_specs=[
            pl.BlockSpec((512, 128), lambda i: (0, 0)),
            pl.BlockSpec((1, 1), lambda i: (0, 0)),
            pl.BlockSpec((1, _BLK), lambda i: (0, i)),
            pl.BlockSpec((_BLK, _D), lambda i: (i, 0)),
            pl.BlockSpec((1, _D), lambda i: (0, 0)),
            pl.BlockSpec((_D, _D), lambda i: (0, 0)),
            pl.BlockSpec((1, _D), lambda i: (0, 0)),
        ],
        out_specs=pl.BlockSpec((1, _D), lambda i: (0, 0)),
        out_shape=jax.ShapeDtypeStruct((1, _D), jnp.float32),
        scratch_shapes=[
            pltpu.VMEM((1, _D), jnp.float32),
            pltpu.SMEM((1,), jnp.int32),
            pltpu.SMEM((1,), jnp.float32),
        ],
        interpret=interpret,
    )(mask2d, sal, wrow, workspace, cont, W, bb)
    return out.reshape(_D)


@jax.jit
def kernel(content, salience, workspace, workspace_mask, W, b):
    return _run(content, salience, workspace, workspace_mask, W, b)


# speedup vs baseline: 3.0742x; 1.4554x over previous
import jax
import jax.numpy as jnp
from jax.experimental import pallas as pl
from jax.experimental.pallas import tpu as pltpu

_CAP = 65536
_D = 512
_BLK = 8192
_GRID = _CAP // _BLK
_BIG_I = 2 ** 30


def _tc_body(mask2d_ref, sal_ref, wrow_ref, ws_ref, content_ref, W_ref, b_ref,
             out_ref, acc_ref, idx_ref, den_ref):
    i = pl.program_id(0)

    @pl.when(i == 0)
    def _stats():
        m = mask2d_ref[...]
        r = jax.lax.broadcasted_iota(jnp.int32, m.shape, 0)
        c = jax.lax.broadcasted_iota(jnp.int32, m.shape, 1)
        j = r * 128 + c
        avail = m < 0.5
        cand = jnp.where(avail, j, _BIG_I)
        first_avail = jnp.min(cand)
        has_avail = first_avail < _BIG_I
        mmin = jnp.min(m)
        idx_min = jnp.min(jnp.where(m == mmin, j, _BIG_I))
        idx = jnp.where(has_avail, first_avail, idx_min)
        em = jnp.exp(m)
        sum_exp = jnp.sum(em)
        exp_at_idx = jnp.sum(jnp.where(j == idx, em, 0.0))
        idx_ref[0] = idx
        den_ref[0] = sum_exp - exp_at_idx
        acc_ref[...] = jnp.zeros_like(acc_ref)

    idx = idx_ref[0]
    w = jnp.exp(wrow_ref[...])
    gl = jax.lax.broadcasted_iota(jnp.int32, w.shape, 1) + i * _BLK
    w = jnp.where(gl == idx, 0.0, w)
    acc_ref[...] += jax.lax.dot_general(
        w, ws_ref[...], (((1,), (0,)), ((), ())),
        preferred_element_type=jnp.float32,
        precision=jax.lax.Precision.DEFAULT)

    @pl.when(i == _GRID - 1)
    def _final():
        es = jnp.exp(sal_ref[...])
        denom = den_ref[0] + es
        bcast = (acc_ref[...] + es * content_ref[...]) / denom
        out_ref[...] = jax.lax.dot_general(
            bcast, W_ref[...], (((1,), (1,)), ((), ())),
            preferred_element_type=jnp.float32,
            precision=jax.lax.Precision.HIGHEST) + b_ref[...]


def _run(content, salience, workspace, workspace_mask, W, b, interpret=False):
    mask2d = workspace_mask.reshape(512, 128)
    wrow = workspace_mask.reshape(1, _CAP)
    sal = salience.reshape(1, 1)
    cont = content.reshape(1, _D)
    bb = b.reshape(1, _D)
    out = pl.pallas_call(
        _tc_body,
        grid=(_GRID,),
        in_specs=[
            pl.BlockSpec((512, 128), lambda i: (0, 0)),
            pl.BlockSpec((1, 1), lambda i: (0, 0)),
            pl.BlockSpec((1, _BLK), lambda i: (0, i)),
            pl.BlockSpec((_BLK, _D), lambda i: (i, 0)),
            pl.BlockSpec((1, _D), lambda i: (0, 0)),
            pl.BlockSpec((_D, _D), lambda i: (0, 0)),
            pl.BlockSpec((1, _D), lambda i: (0, 0)),
        ],
        out_specs=pl.BlockSpec((1, _D), lambda i: (0, 0)),
        out_shape=jax.ShapeDtypeStruct((1, _D), jnp.float32),
        scratch_shapes=[
            pltpu.VMEM((1, _D), jnp.float32),
            pltpu.SMEM((1,), jnp.int32),
            pltpu.SMEM((1,), jnp.float32),
        ],
        interpret=interpret,
    )(mask2d, sal, wrow, workspace, cont, W, bb)
    return out.reshape(_D)


@jax.jit
def kernel(content, salience, workspace, workspace_mask, W, b):
    return _run(content, salience, workspace, workspace_mask, W, b)
